# baseline (speedup 1.0000x reference)
import jax
import jax.numpy as jnp
from jax.experimental import pallas as pl

E = 160000
CTX = 5
HID = 128
OUT_DIM = 40
BE = 2000


def _mlp_body(x_ref, et_ref, wc_ref, bc_ref, t2_ref, w2_ref, b2_ref,
              w3_ref, b3_ref, o_ref):
    x = x_ref[...]
    et = et_ref[...]
    wc = wc_ref[...]
    t2 = t2_ref[...]
    z1 = bc_ref[...] + x[:, 0:1] * wc[0:1, :]
    for c in range(1, CTX):
        z1 = z1 + x[:, c:c + 1] * wc[c:c + 1, :]
    for t in range(15):
        z1 = z1 + jnp.where(et == t, 1.0, 0.0) * t2[t:t + 1, :]
    h1 = z1 * jax.nn.sigmoid(z1)
    z2 = jnp.dot(h1, w2_ref[...], preferred_element_type=jnp.float32) + b2_ref[...]
    h2 = z2 * jax.nn.sigmoid(z2)
    o_ref[...] = jnp.dot(h2, w3_ref[...], preferred_element_type=jnp.float32) + b3_ref[...]


def kernel(edge_attr, edge_type, layer_embed_w, W_proj, b_proj,
           W1, b1, W2, b2, W3, b3, k=1):
    e = edge_attr.shape[0]
    wc = jnp.dot(W1, W_proj)
    wc_t = jnp.zeros((8, HID), jnp.float32).at[:CTX, :].set(wc.T)
    bc = (jnp.dot(W1, b_proj) + b1).reshape(1, HID)
    t2 = jnp.dot(layer_embed_w, W1.T)
    t2p = jnp.zeros((16, HID), jnp.float32).at[:15, :].set(t2)
    w2_t = W2.T
    b2r = b2.reshape(1, HID)
    w3_t = W3.T
    b3r = b3.reshape(1, OUT_DIM)
    x = jnp.pad(edge_attr, ((0, 0), (0, 8 - CTX)))
    et = edge_type.astype(jnp.int32).reshape(e, 1)

    grid = (e // BE,)
    rep = lambda i: (0, 0)
    out = pl.pallas_call(
        _mlp_body,
        grid=grid,
        in_specs=[
            pl.BlockSpec((BE, 8), lambda i: (i, 0)),
            pl.BlockSpec((BE, 1), lambda i: (i, 0)),
            pl.BlockSpec((8, HID), rep),
            pl.BlockSpec((1, HID), rep),
            pl.BlockSpec((16, HID), rep),
            pl.BlockSpec((HID, HID), rep),
            pl.BlockSpec((1, HID), rep),
            pl.BlockSpec((HID, OUT_DIM), rep),
            pl.BlockSpec((1, OUT_DIM), rep),
        ],
        out_specs=pl.BlockSpec((BE, OUT_DIM), lambda i: (i, 0)),
        out_shape=jax.ShapeDtypeStruct((e, OUT_DIM), jnp.float32),
    )(x, et, wc_t, bc, t2p, w2_t, b2r, w3_t, b3r)
    return out.reshape(e, 1, OUT_DIM)


# speedup vs baseline: 1.7039x; 1.7039x over previous
import jax
import jax.numpy as jnp
from jax.experimental import pallas as pl

E = 160000
CTX = 5
HID = 128
OUT_DIM = 40
BE = 2000


def _mlp_body(x_ref, et_ref, wc_ref, bc_ref, t2_ref, w2_ref, b2_ref,
              w3_ref, b3_ref, o_ref):
    x = x_ref[...]
    et = et_ref[...]
    onehot = (et == jax.lax.broadcasted_iota(jnp.int32, (1, 16), 1)
              ).astype(jnp.float32)
    z1 = jnp.dot(x, wc_ref[...], preferred_element_type=jnp.float32)
    z1 = z1 + jnp.dot(onehot, t2_ref[...], preferred_element_type=jnp.float32)
    z1 = z1 + bc_ref[...]
    h1 = z1 * jax.nn.sigmoid(z1)
    z2 = jnp.dot(h1, w2_ref[...], preferred_element_type=jnp.float32) + b2_ref[...]
    h2 = z2 * jax.nn.sigmoid(z2)
    o_ref[...] = jnp.dot(h2, w3_ref[...], preferred_element_type=jnp.float32) + b3_ref[...]


def kernel(edge_attr, edge_type, layer_embed_w, W_proj, b_proj,
           W1, b1, W2, b2, W3, b3, k=1):
    e = edge_attr.shape[0]
    wc = jnp.dot(W1, W_proj)
    wc_t = jnp.zeros((8, HID), jnp.float32).at[:CTX, :].set(wc.T)
    bc = (jnp.dot(W1, b_proj) + b1).reshape(1, HID)
    t2 = jnp.dot(layer_embed_w, W1.T)
    t2p = jnp.zeros((16, HID), jnp.float32).at[:15, :].set(t2)
    w2_t = W2.T
    b2r = b2.reshape(1, HID)
    w3_t = W3.T
    b3r = b3.reshape(1, OUT_DIM)
    x = jnp.pad(edge_attr, ((0, 0), (0, 8 - CTX)))
    et = edge_type.astype(jnp.int32).reshape(e, 1)

    grid = (e // BE,)
    rep = lambda i: (0, 0)
    out = pl.pallas_call(
        _mlp_body,
        grid=grid,
        in_specs=[
            pl.BlockSpec((BE, 8), lambda i: (i, 0)),
            pl.BlockSpec((BE, 1), lambda i: (i, 0)),
            pl.BlockSpec((8, HID), rep),
            pl.BlockSpec((1, HID), rep),
            pl.BlockSpec((16, HID), rep),
            pl.BlockSpec((HID, HID), rep),
            pl.BlockSpec((1, HID), rep),
            pl.BlockSpec((HID, OUT_DIM), rep),
            pl.BlockSpec((1, OUT_DIM), rep),
        ],
        out_specs=pl.BlockSpec((BE, OUT_DIM), lambda i: (i, 0)),
        out_shape=jax.ShapeDtypeStruct((e, OUT_DIM), jnp.float32),
    )(x, et, wc_t, bc, t2p, w2_t, b2r, w3_t, b3r)
    return out.reshape(e, 1, OUT_DIM)


# speedup vs baseline: 2.6764x; 1.5707x over previous
import jax
import jax.numpy as jnp
from jax.experimental import pallas as pl

E = 160000
CTX = 5
HID = 128
OUT_DIM = 40
BE = 2000


def _mlp_body(x_ref, wc_ref, bc_ref, t2_ref, w2_ref, b2_ref,
              w3_ref, b3_ref, o_ref):
    x = x_ref[...]
    et = x[:, CTX:CTX + 1].astype(jnp.int32)
    onehot = (et == jax.lax.broadcasted_iota(jnp.int32, (1, 16), 1)
              ).astype(jnp.float32)
    z1 = jnp.dot(x, wc_ref[...], preferred_element_type=jnp.float32)
    z1 = z1 + jnp.dot(onehot, t2_ref[...], preferred_element_type=jnp.float32)
    z1 = z1 + bc_ref[...]
    h1 = z1 * jax.nn.sigmoid(z1)
    z2 = jnp.dot(h1, w2_ref[...], preferred_element_type=jnp.float32) + b2_ref[...]
    h2 = z2 * jax.nn.sigmoid(z2)
    o_ref[...] = jnp.dot(h2, w3_ref[...], preferred_element_type=jnp.float32) + b3_ref[...]


def kernel(edge_attr, edge_type, layer_embed_w, W_proj, b_proj,
           W1, b1, W2, b2, W3, b3, k=1):
    e = edge_attr.shape[0]
    wc = jnp.dot(W1, W_proj)
    wc_t = jnp.zeros((8, HID), jnp.float32).at[:CTX, :].set(wc.T)
    bc = (jnp.dot(W1, b_proj) + b1).reshape(1, HID)
    t2 = jnp.dot(layer_embed_w, W1.T)
    t2p = jnp.zeros((16, HID), jnp.float32).at[:15, :].set(t2)
    w2_t = W2.T
    b2r = b2.reshape(1, HID)
    w3_t = W3.T
    b3r = b3.reshape(1, OUT_DIM)
    x = jnp.concatenate(
        [edge_attr,
         edge_type.astype(jnp.float32)[:, None],
         jnp.zeros((e, 2), jnp.float32)], axis=1)

    grid = (e // BE,)
    rep = lambda i: (0, 0)
    out = pl.pallas_call(
        _mlp_body,
        grid=grid,
        in_specs=[
            pl.BlockSpec((BE, 8), lambda i: (i, 0)),
            pl.BlockSpec((8, HID), rep),
            pl.BlockSpec((1, HID), rep),
            pl.BlockSpec((16, HID), rep),
            pl.BlockSpec((HID, HID), rep),
            pl.BlockSpec((1, HID), rep),
            pl.BlockSpec((HID, OUT_DIM), rep),
            pl.BlockSpec((1, OUT_DIM), rep),
        ],
        out_specs=pl.BlockSpec((BE, OUT_DIM), lambda i: (i, 0)),
        out_shape=jax.ShapeDtypeStruct((e, OUT_DIM), jnp.float32),
    )(x, wc_t, bc, t2p, w2_t, b2r, w3_t, b3r)
    return out.reshape(e, 1, OUT_DIM)


# speedup vs baseline: 2.9997x; 1.1208x over previous
import jax
import jax.numpy as jnp
from jax.experimental import pallas as pl

E = 160000
CTX = 5
HID = 128
OUT_DIM = 40
BE = 4000


def _mlp_body(x_ref, wc_ref, bc_ref, t2_ref, w2_ref, b2_ref,
              w3_ref, b3_ref, o_ref):
    x = x_ref[...]
    et = x[:, CTX:CTX + 1].astype(jnp.int32)
    onehot = (et == jax.lax.broadcasted_iota(jnp.int32, (1, 16), 1)
              ).astype(jnp.float32)
    z1 = jnp.dot(x, wc_ref[...], preferred_element_type=jnp.float32)
    z1 = z1 + jnp.dot(onehot, t2_ref[...], preferred_element_type=jnp.float32)
    z1 = z1 + bc_ref[...]
    h1 = z1 * jax.nn.sigmoid(z1)
    z2 = jnp.dot(h1, w2_ref[...], preferred_element_type=jnp.float32) + b2_ref[...]
    h2 = z2 * jax.nn.sigmoid(z2)
    o_ref[...] = jnp.dot(h2, w3_ref[...], preferred_element_type=jnp.float32) + b3_ref[...]


def kernel(edge_attr, edge_type, layer_embed_w, W_proj, b_proj,
           W1, b1, W2, b2, W3, b3, k=1):
    e = edge_attr.shape[0]
    wc = jnp.dot(W1, W_proj)
    wc_t = jnp.zeros((8, HID), jnp.float32).at[:CTX, :].set(wc.T)
    bc = (jnp.dot(W1, b_proj) + b1).reshape(1, HID)
    t2 = jnp.dot(layer_embed_w, W1.T)
    t2p = jnp.zeros((16, HID), jnp.float32).at[:15, :].set(t2)
    w2_t = W2.T
    b2r = b2.reshape(1, HID)
    w3_t = W3.T
    b3r = b3.reshape(1, OUT_DIM)
    x = jnp.concatenate(
        [edge_attr,
         edge_type.astype(jnp.float32)[:, None],
         jnp.zeros((e, 2), jnp.float32)], axis=1)

    grid = (e // BE,)
    rep = lambda i: (0, 0)
    out = pl.pallas_call(
        _mlp_body,
        grid=grid,
        in_specs=[
            pl.BlockSpec((BE, 8), lambda i: (i, 0)),
            pl.BlockSpec((8, HID), rep),
            pl.BlockSpec((1, HID), rep),
            pl.BlockSpec((16, HID), rep),
            pl.BlockSpec((HID, HID), rep),
            pl.BlockSpec((1, HID), rep),
            pl.BlockSpec((HID, OUT_DIM), rep),
            pl.BlockSpec((1, OUT_DIM), rep),
        ],
        out_specs=pl.BlockSpec((BE, OUT_DIM), lambda i: (i, 0)),
        out_shape=jax.ShapeDtypeStruct((e, OUT_DIM), jnp.float32),
    )(x, wc_t, bc, t2p, w2_t, b2r, w3_t, b3r)
    return out.reshape(e, 1, OUT_DIM)


# speedup vs baseline: 3.0958x; 1.0320x over previous
import jax
import jax.numpy as jnp
from jax.experimental import pallas as pl

E = 160000
CTX = 5
HID = 128
OUT_DIM = 40
BE = 8000


def _mlp_body(x_ref, wc_ref, bc_ref, t2_ref, w2_ref, b2_ref,
              w3_ref, b3_ref, o_ref):
    x = x_ref[...]
    et = x[:, CTX:CTX + 1].astype(jnp.int32)
    onehot = (et == jax.lax.broadcasted_iota(jnp.int32, (1, 16), 1)
              ).astype(jnp.float32)
    z1 = jnp.dot(x, wc_ref[...], preferred_element_type=jnp.float32)
    z1 = z1 + jnp.dot(onehot, t2_ref[...], preferred_element_type=jnp.float32)
    z1 = z1 + bc_ref[...]
    h1 = z1 * jax.nn.sigmoid(z1)
    z2 = jnp.dot(h1, w2_ref[...], preferred_element_type=jnp.float32) + b2_ref[...]
    h2 = z2 * jax.nn.sigmoid(z2)
    o_ref[...] = jnp.dot(h2, w3_ref[...], preferred_element_type=jnp.float32) + b3_ref[...]


def kernel(edge_attr, edge_type, layer_embed_w, W_proj, b_proj,
           W1, b1, W2, b2, W3, b3, k=1):
    e = edge_attr.shape[0]
    wc = jnp.dot(W1, W_proj)
    wc_t = jnp.zeros((8, HID), jnp.float32).at[:CTX, :].set(wc.T)
    bc = (jnp.dot(W1, b_proj) + b1).reshape(1, HID)
    t2 = jnp.dot(layer_embed_w, W1.T)
    t2p = jnp.zeros((16, HID), jnp.float32).at[:15, :].set(t2)
    w2_t = W2.T
    b2r = b2.reshape(1, HID)
    w3_t = W3.T
    b3r = b3.reshape(1, OUT_DIM)
    x = jnp.concatenate(
        [edge_attr,
         edge_type.astype(jnp.float32)[:, None],
         jnp.zeros((e, 2), jnp.float32)], axis=1)

    grid = (e // BE,)
    rep = lambda i: (0, 0)
    out = pl.pallas_call(
        _mlp_body,
        grid=grid,
        in_specs=[
            pl.BlockSpec((BE, 8), lambda i: (i, 0)),
            pl.BlockSpec((8, HID), rep),
            pl.BlockSpec((1, HID), rep),
            pl.BlockSpec((16, HID), rep),
            pl.BlockSpec((HID, HID), rep),
            pl.BlockSpec((1, HID), rep),
            pl.BlockSpec((HID, OUT_DIM), rep),
            pl.BlockSpec((1, OUT_DIM), rep),
        ],
        out_specs=pl.BlockSpec((BE, OUT_DIM), lambda i: (i, 0)),
        out_shape=jax.ShapeDtypeStruct((e, OUT_DIM), jnp.float32),
    )(x, wc_t, bc, t2p, w2_t, b2r, w3_t, b3r)
    return out.reshape(e, 1, OUT_DIM)


# speedup vs baseline: 3.1048x; 1.0029x over previous
import jax
import jax.numpy as jnp
from jax.experimental import pallas as pl

E = 160000
CTX = 5
HID = 128
OUT_DIM = 40
BE = 16000


def _mlp_body(x_ref, wc_ref, bc_ref, t2_ref, w2_ref, b2_ref,
              w3_ref, b3_ref, o_ref):
    x = x_ref[...]
    et = x[:, CTX:CTX + 1].astype(jnp.int32)
    onehot = (et == jax.lax.broadcasted_iota(jnp.int32, (1, 16), 1)
              ).astype(jnp.float32)
    z1 = jnp.dot(x, wc_ref[...], preferred_element_type=jnp.float32)
    z1 = z1 + jnp.dot(onehot, t2_ref[...], preferred_element_type=jnp.float32)
    z1 = z1 + bc_ref[...]
    h1 = z1 * jax.nn.sigmoid(z1)
    z2 = jnp.dot(h1, w2_ref[...], preferred_element_type=jnp.float32) + b2_ref[...]
    h2 = z2 * jax.nn.sigmoid(z2)
    o_ref[...] = jnp.dot(h2, w3_ref[...], preferred_element_type=jnp.float32) + b3_ref[...]


def kernel(edge_attr, edge_type, layer_embed_w, W_proj, b_proj,
           W1, b1, W2, b2, W3, b3, k=1):
    e = edge_attr.shape[0]
    wc = jnp.dot(W1, W_proj)
    wc_t = jnp.zeros((8, HID), jnp.float32).at[:CTX, :].set(wc.T)
    bc = (jnp.dot(W1, b_proj) + b1).reshape(1, HID)
    t2 = jnp.dot(layer_embed_w, W1.T)
    t2p = jnp.zeros((16, HID), jnp.float32).at[:15, :].set(t2)
    w2_t = W2.T
    b2r = b2.reshape(1, HID)
    w3_t = W3.T
    b3r = b3.reshape(1, OUT_DIM)
    x = jnp.concatenate(
        [edge_attr,
         edge_type.astype(jnp.float32)[:, None],
         jnp.zeros((e, 2), jnp.float32)], axis=1)

    grid = (e // BE,)
    rep = lambda i: (0, 0)
    out = pl.pallas_call(
        _mlp_body,
        grid=grid,
        in_specs=[
            pl.BlockSpec((BE, 8), lambda i: (i, 0)),
            pl.BlockSpec((8, HID), rep),
            pl.BlockSpec((1, HID), rep),
            pl.BlockSpec((16, HID), rep),
            pl.BlockSpec((HID, HID), rep),
            pl.BlockSpec((1, HID), rep),
            pl.BlockSpec((HID, OUT_DIM), rep),
            pl.BlockSpec((1, OUT_DIM), rep),
        ],
        out_specs=pl.BlockSpec((BE, OUT_DIM), lambda i: (i, 0)),
        out_shape=jax.ShapeDtypeStruct((e, OUT_DIM), jnp.float32),
    )(x, wc_t, bc, t2p, w2_t, b2r, w3_t, b3r)
    return out.reshape(e, 1, OUT_DIM)


# speedup vs baseline: 3.2711x; 1.0536x over previous
import jax
import jax.numpy as jnp
from jax.experimental import pallas as pl

E = 160000
CTX = 5
HID = 128
OUT_DIM = 40
BE = 16000


def _mlp_body(x_ref, wc_ref, bc_ref, t2_ref, w2_ref, b2_ref,
              w3_ref, b3_ref, o_ref):
    x = x_ref[...]
    et = x[:, CTX:CTX + 1].astype(jnp.int32)
    onehot = (et == jax.lax.broadcasted_iota(jnp.int32, (1, 16), 1)
              ).astype(jnp.bfloat16)
    z1 = jnp.dot(x, wc_ref[...].astype(jnp.bfloat16),
                 preferred_element_type=jnp.float32)
    z1 = z1 + jnp.dot(onehot, t2_ref[...], preferred_element_type=jnp.float32)
    z1 = z1 + bc_ref[...]
    h1 = (z1 * jax.nn.sigmoid(z1)).astype(jnp.bfloat16)
    z2 = jnp.dot(h1, w2_ref[...], preferred_element_type=jnp.float32) + b2_ref[...]
    h2 = (z2 * jax.nn.sigmoid(z2)).astype(jnp.bfloat16)
    o_ref[...] = jnp.dot(h2, w3_ref[...], preferred_element_type=jnp.float32) + b3_ref[...]


def kernel(edge_attr, edge_type, layer_embed_w, W_proj, b_proj,
           W1, b1, W2, b2, W3, b3, k=1):
    e = edge_attr.shape[0]
    wc = jnp.dot(W1, W_proj)
    wc_t = jnp.zeros((8, HID), jnp.float32).at[:CTX, :].set(wc.T)
    bc = (jnp.dot(W1, b_proj) + b1).reshape(1, HID)
    t2 = jnp.dot(layer_embed_w, W1.T)
    t2p = jnp.zeros((16, HID), jnp.bfloat16).at[:15, :].set(t2.astype(jnp.bfloat16))
    w2_t = W2.T.astype(jnp.bfloat16)
    b2r = b2.reshape(1, HID)
    w3_t = W3.T.astype(jnp.bfloat16)
    b3r = b3.reshape(1, OUT_DIM)
    x = jnp.concatenate(
        [edge_attr.astype(jnp.bfloat16),
         edge_type.astype(jnp.bfloat16)[:, None],
         jnp.zeros((e, 2), jnp.bfloat16)], axis=1)

    grid = (e // BE,)
    rep = lambda i: (0, 0)
    out = pl.pallas_call(
        _mlp_body,
        grid=grid,
        in_specs=[
            pl.BlockSpec((BE, 8), lambda i: (i, 0)),
            pl.BlockSpec((8, HID), rep),
            pl.BlockSpec((1, HID), rep),
            pl.BlockSpec((16, HID), rep),
            pl.BlockSpec((HID, HID), rep),
            pl.BlockSpec((1, HID), rep),
            pl.BlockSpec((HID, OUT_DIM), rep),
            pl.BlockSpec((1, OUT_DIM), rep),
        ],
        out_specs=pl.BlockSpec((BE, OUT_DIM), lambda i: (i, 0)),
        out_shape=jax.ShapeDtypeStruct((e, OUT_DIM), jnp.float32),
    )(x, wc_t, bc, t2p, w2_t, b2r, w3_t, b3r)
    return out.reshape(e, 1, OUT_DIM)


# speedup vs baseline: 3.6892x; 1.1278x over previous
import jax
import jax.numpy as jnp
from jax.experimental import pallas as pl

E = 160000
CTX = 5
HID = 128
OUT_DIM = 40
BE = 16000


def _mlp_body(x_ref, wc_ref, t2_ref, w2_ref, b2_ref,
              w3_ref, b3_ref, o_ref):
    x = x_ref[...]
    et = x[:, CTX:CTX + 1].astype(jnp.int32)
    onehot = (et == jax.lax.broadcasted_iota(jnp.int32, (1, 16), 1)
              ).astype(jnp.bfloat16)
    z1 = jnp.dot(x, wc_ref[...].astype(jnp.bfloat16),
                 preferred_element_type=jnp.float32)
    z1 = (z1 + jnp.dot(onehot, t2_ref[...], preferred_element_type=jnp.float32)
          ).astype(jnp.bfloat16)
    h1 = z1 * jax.nn.sigmoid(z1)
    z2 = (jnp.dot(h1, w2_ref[...], preferred_element_type=jnp.float32)
          + b2_ref[...]).astype(jnp.bfloat16)
    h2 = z2 * jax.nn.sigmoid(z2)
    o_ref[...] = jnp.dot(h2, w3_ref[...], preferred_element_type=jnp.float32) + b3_ref[...]


def kernel(edge_attr, edge_type, layer_embed_w, W_proj, b_proj,
           W1, b1, W2, b2, W3, b3, k=1):
    e = edge_attr.shape[0]
    wc = jnp.dot(W1, W_proj)
    wc_t = jnp.zeros((8, HID), jnp.float32).at[:CTX, :].set(wc.T)
    bc = (jnp.dot(W1, b_proj) + b1).reshape(1, HID)
    t2 = jnp.dot(layer_embed_w, W1.T) + bc
    t2p = jnp.zeros((16, HID), jnp.bfloat16).at[:15, :].set(t2.astype(jnp.bfloat16))
    w2_t = W2.T.astype(jnp.bfloat16)
    b2r = b2.reshape(1, HID)
    w3_t = W3.T.astype(jnp.bfloat16)
    b3r = b3.reshape(1, OUT_DIM)
    x = jnp.concatenate(
        [edge_attr.astype(jnp.bfloat16),
         edge_type.astype(jnp.bfloat16)[:, None],
         jnp.zeros((e, 2), jnp.bfloat16)], axis=1)

    grid = (e // BE,)
    rep = lambda i: (0, 0)
    out = pl.pallas_call(
        _mlp_body,
        grid=grid,
        in_specs=[
            pl.BlockSpec((BE, 8), lambda i: (i, 0)),
            pl.BlockSpec((8, HID), rep),
            pl.BlockSpec((16, HID), rep),
            pl.BlockSpec((HID, HID), rep),
            pl.BlockSpec((1, HID), rep),
            pl.BlockSpec((HID, OUT_DIM), rep),
            pl.BlockSpec((1, OUT_DIM), rep),
        ],
        out_specs=pl.BlockSpec((BE, OUT_DIM), lambda i: (i, 0)),
        out_shape=jax.ShapeDtypeStruct((e, OUT_DIM), jnp.float32),
    )(x, wc_t, t2p, w2_t, b2r, w3_t, b3r)
    return out.reshape(e, 1, OUT_DIM)


# speedup vs baseline: 3.8887x; 1.0541x over previous
import jax
import jax.numpy as jnp
from jax.experimental import pallas as pl

E = 160000
CTX = 5
HID = 128
OUT_DIM = 40
N_TYPES = 15
KIN = 24
BE = 16000


def _mlp_body(x_ref, wcat_ref, w2_ref, b2_ref, w3_ref, b3_ref, o_ref):
    x = x_ref[...]
    z1 = jnp.dot(x, wcat_ref[...].astype(jnp.bfloat16),
                 preferred_element_type=jnp.float32).astype(jnp.bfloat16)
    h1 = z1 * jax.nn.sigmoid(z1)
    z2 = (jnp.dot(h1, w2_ref[...], preferred_element_type=jnp.float32)
          ).astype(jnp.bfloat16) + b2_ref[...]
    h2 = z2 * jax.nn.sigmoid(z2)
    o_ref[...] = jnp.dot(h2, w3_ref[...],
                         preferred_element_type=jnp.float32) + b3_ref[...]


def kernel(edge_attr, edge_type, layer_embed_w, W_proj, b_proj,
           W1, b1, W2, b2, W3, b3, k=1):
    e = edge_attr.shape[0]
    wc = jnp.dot(W1, W_proj)
    bc = (jnp.dot(W1, b_proj) + b1).reshape(1, HID)
    t2 = jnp.dot(layer_embed_w, W1.T) + bc
    wcat = jnp.zeros((KIN, HID), jnp.float32)
    wcat = wcat.at[:CTX, :].set(wc.T)
    wcat = wcat.at[CTX:CTX + N_TYPES, :].set(t2)
    w2_t = W2.T.astype(jnp.bfloat16)
    b2r = b2.reshape(1, HID).astype(jnp.bfloat16)
    w3_t = W3.T.astype(jnp.bfloat16)
    b3r = b3.reshape(1, OUT_DIM)
    onehot = (edge_type.astype(jnp.int32)[:, None]
              == jnp.arange(N_TYPES, dtype=jnp.int32)[None, :])
    x = jnp.concatenate(
        [edge_attr.astype(jnp.bfloat16),
         onehot.astype(jnp.bfloat16),
         jnp.zeros((e, KIN - CTX - N_TYPES), jnp.bfloat16)], axis=1)

    grid = (e // BE,)
    rep = lambda i: (0, 0)
    out = pl.pallas_call(
        _mlp_body,
        grid=grid,
        in_specs=[
            pl.BlockSpec((BE, KIN), lambda i: (i, 0)),
            pl.BlockSpec((KIN, HID), rep),
            pl.BlockSpec((HID, HID), rep),
            pl.BlockSpec((1, HID), rep),
            pl.BlockSpec((HID, OUT_DIM), rep),
            pl.BlockSpec((1, OUT_DIM), rep),
        ],
        out_specs=pl.BlockSpec((BE, OUT_DIM), lambda i: (i, 0)),
        out_shape=jax.ShapeDtypeStruct((e, OUT_DIM), jnp.float32),
    )(x, wcat, w2_t, b2r, w3_t, b3r)
    return out.reshape(e, 1, OUT_DIM)
